# baseline (speedup 1.0000x reference)
import functools

import jax
import jax.numpy as jnp
from jax import lax
from jax.experimental import pallas as pl
from jax.experimental.pallas import tpu as pltpu
from jax.experimental.pallas import tpu_sc as plsc

NUM_Z = 4
Z_VOCAB = 100000
NUM_B = 4
B_VOCAB = 1000
ED = 64
IMG = 512
BATCH = 16384

NC = 2
NS = 16
NW = NC * NS
BPW = BATCH // NW
CH = 128
NCH = BPW // CH
NARM = NUM_Z + NUM_B
PHI_BLKS = IMG // ED


def _body(idx_hbm, phi_hbm, zf_hbm, bf_hbm, out_hbm, idx_v, rows_v, phi_v, sem):
    wid = lax.axis_index("s") * NC + lax.axis_index("c")
    base = wid * BPW

    pltpu.sync_copy(idx_hbm.at[wid], idx_v)

    for a in range(NARM):
        table = zf_hbm if a < NUM_Z else bf_hbm
        for c in range(NCH):
            buf = c % 2
            pltpu.async_copy(table.at[idx_v.at[a, c]], rows_v.at[buf], sem).wait()
            pltpu.sync_copy(
                rows_v.at[buf],
                out_hbm.at[pl.ds(base + c * CH, CH), a],
            )

    for c in range(NCH):
        pltpu.sync_copy(phi_hbm.at[pl.ds(base + c * CH, CH)], phi_v)
        pltpu.sync_copy(phi_v, out_hbm.at[pl.ds(base + c * CH, CH), pl.ds(NARM, PHI_BLKS)])


@jax.jit
def _run(idx, phi_r, z_flat, b_flat):
    mesh = plsc.VectorSubcoreMesh(
        core_axis_name="c", subcore_axis_name="s", num_cores=NC, num_subcores=NS
    )
    return pl.kernel(
        _body,
        out_type=jax.ShapeDtypeStruct((BATCH, NARM + PHI_BLKS, ED), jnp.float32),
        mesh=mesh,
        scratch_types=[
            pltpu.VMEM((NARM, NCH, CH), jnp.int32),
            pltpu.VMEM((2, CH, ED), jnp.float32),
            pltpu.VMEM((CH, PHI_BLKS, ED), jnp.float32),
            pltpu.SemaphoreType.DMA,
        ],
        compiler_params=pltpu.CompilerParams(use_tc_tiling_on_sc=False),
    )(idx, phi_r, z_flat, b_flat)


def kernel(z, beta, phi_x, Z_tables, B_tables):
    zoff = jnp.arange(NUM_Z, dtype=jnp.int32) * Z_VOCAB
    boff = jnp.arange(NUM_B, dtype=jnp.int32) * B_VOCAB
    zi = (z.astype(jnp.int32) + zoff[None, :]).T
    bi = (beta.astype(jnp.int32) + boff[None, :]).T
    idx = jnp.concatenate([zi, bi], axis=0)
    idx = (
        idx.reshape(NARM, NW, NCH * CH).transpose(1, 0, 2).reshape(NW, NARM, NCH, CH)
    )
    out = _run(
        idx,
        phi_x.reshape(BATCH, PHI_BLKS, ED),
        Z_tables.reshape(NUM_Z * Z_VOCAB, ED),
        B_tables.reshape(NUM_B * B_VOCAB, ED),
    )
    return out.reshape(BATCH, (NARM + PHI_BLKS) * ED)
